# initial kernel scaffold (unmeasured)
import jax
import jax.numpy as jnp
from jax import lax
from jax.experimental import pallas as pl
from jax.experimental.pallas import tpu as pltpu

N_DEV = 4


def _a2a_body(x_ref, out_ref, send_sems, recv_sems, local_sem):
    my_i = lax.axis_index("i")
    m = out_ref.shape[0]

    barrier_sem = pltpu.get_barrier_semaphore()
    for k in range(1, N_DEV):
        peer = lax.rem(my_i + k, N_DEV)
        pl.semaphore_signal(
            barrier_sem, inc=1,
            device_id=(peer,), device_id_type=pl.DeviceIdType.MESH,
        )
    pl.semaphore_wait(barrier_sem, N_DEV - 1)

    local_copy = pltpu.make_async_copy(
        x_ref.at[pl.ds(my_i * m, m), :],
        out_ref.at[:, pl.ds(my_i * m, m)],
        local_sem,
    )
    local_copy.start()

    rdmas = []
    for k in range(1, N_DEV):
        dst = lax.rem(my_i + k, N_DEV)
        rdma = pltpu.make_async_remote_copy(
            src_ref=x_ref.at[pl.ds(dst * m, m), :],
            dst_ref=out_ref.at[:, pl.ds(my_i * m, m)],
            send_sem=send_sems.at[k - 1],
            recv_sem=recv_sems.at[k - 1],
            device_id=(dst,),
            device_id_type=pl.DeviceIdType.MESH,
        )
        rdma.start()
        rdmas.append(rdma)

    for k in range(1, N_DEV):
        src = lax.rem(my_i - k + N_DEV, N_DEV)
        recv = pltpu.make_async_remote_copy(
            src_ref=x_ref.at[pl.ds(src * m, m), :],
            dst_ref=out_ref.at[:, pl.ds(src * m, m)],
            send_sem=send_sems.at[k - 1],
            recv_sem=recv_sems.at[k - 1],
            device_id=(src,),
            device_id_type=pl.DeviceIdType.MESH,
        )
        recv.wait_recv()

    for rdma in rdmas:
        rdma.wait_send()
    local_copy.wait()


def kernel(x, w_mat):
    k_total, m = x.shape

    xg = pl.pallas_call(
        _a2a_body,
        out_shape=jax.ShapeDtypeStruct((m, k_total), x.dtype),
        in_specs=[pl.BlockSpec(memory_space=pltpu.ANY)],
        out_specs=pl.BlockSpec(memory_space=pltpu.ANY),
        scratch_shapes=[
            pltpu.SemaphoreType.DMA((N_DEV - 1,)),
            pltpu.SemaphoreType.DMA((N_DEV - 1,)),
            pltpu.SemaphoreType.DMA,
        ],
        compiler_params=pltpu.CompilerParams(collective_id=0),
    )(x)

    y = jnp.dot(xg, w_mat, preferred_element_type=jnp.float32)
    return jnp.maximum(y, 0.0)


# baseline (device time: 704716 ns/iter reference)
import jax
import jax.numpy as jnp
from jax import lax
from jax.experimental import pallas as pl
from jax.experimental.pallas import tpu as pltpu

N_DEV = 4


def _a2a_body(x_ref, out_ref, send_sems, recv_sems, local_sem):
    my_i = lax.axis_index("i")
    m = out_ref.shape[0]

    barrier_sem = pltpu.get_barrier_semaphore()
    for k in range(1, N_DEV):
        peer = lax.rem(my_i + k, N_DEV)
        pl.semaphore_signal(
            barrier_sem, inc=1,
            device_id=(peer,), device_id_type=pl.DeviceIdType.MESH,
        )
    pl.semaphore_wait(barrier_sem, N_DEV - 1)

    local_copy = pltpu.make_async_copy(
        x_ref.at[pl.ds(my_i * m, m), :],
        out_ref.at[:, pl.ds(my_i * m, m)],
        local_sem,
    )
    local_copy.start()

    rdmas = []
    for k in range(1, N_DEV):
        dst = lax.rem(my_i + k, N_DEV)
        rdma = pltpu.make_async_remote_copy(
            src_ref=x_ref.at[pl.ds(dst * m, m), :],
            dst_ref=out_ref.at[:, pl.ds(my_i * m, m)],
            send_sem=send_sems.at[k - 1],
            recv_sem=recv_sems.at[k - 1],
            device_id=(dst,),
            device_id_type=pl.DeviceIdType.MESH,
        )
        rdma.start()
        rdmas.append(rdma)

    for k in range(1, N_DEV):
        src = lax.rem(my_i - k + N_DEV, N_DEV)
        recv = pltpu.make_async_remote_copy(
            src_ref=x_ref.at[pl.ds(src * m, m), :],
            dst_ref=out_ref.at[:, pl.ds(src * m, m)],
            send_sem=send_sems.at[k - 1],
            recv_sem=recv_sems.at[k - 1],
            device_id=(src,),
            device_id_type=pl.DeviceIdType.MESH,
        )
        recv.wait_recv()

    for rdma in rdmas:
        rdma.wait_send()
    local_copy.wait()


def kernel(x, w_mat):
    k_total, m = x.shape

    xg = pl.pallas_call(
        _a2a_body,
        out_shape=jax.ShapeDtypeStruct((m, k_total), x.dtype),
        in_specs=[pl.BlockSpec(memory_space=pl.ANY)],
        out_specs=pl.BlockSpec(memory_space=pl.ANY),
        scratch_shapes=[
            pltpu.SemaphoreType.DMA((N_DEV - 1,)),
            pltpu.SemaphoreType.DMA((N_DEV - 1,)),
            pltpu.SemaphoreType.DMA,
        ],
        compiler_params=pltpu.CompilerParams(collective_id=0),
    )(x)

    y = jnp.dot(xg, w_mat, preferred_element_type=jnp.float32)
    return jnp.maximum(y, 0.0)


# device time: 471216 ns/iter; 1.4955x vs baseline; 1.4955x over previous
import jax

try:
    jax.config.update("jax_compilation_cache_dir", "/tmp/jax_pallas_cache")
    jax.config.update("jax_persistent_cache_min_entry_size_bytes", -1)
    jax.config.update("jax_persistent_cache_min_compile_time_secs", 0.0)
except Exception:
    pass

import jax.numpy as jnp
from jax import lax
from jax.experimental import pallas as pl
from jax.experimental.pallas import tpu as pltpu

N_DEV = 4
M = 2048
CH = 4
KC = M // CH


def _body(x_ref, w_ref, out_ref, slots_ref,
          acc_ref, xv_ref, wv_ref,
          send_sems, recv_sems, dma_sems, out_sem):
    my_i = lax.axis_index("i")

    barrier_sem = pltpu.get_barrier_semaphore()
    for k in range(1, N_DEV):
        peer = lax.rem(my_i + k, N_DEV)
        pl.semaphore_signal(
            barrier_sem, inc=1,
            device_id=(peer,), device_id_type=pl.DeviceIdType.MESH,
        )
    pl.semaphore_wait(barrier_sem, N_DEV - 1)

    sends = []
    for k in range(1, N_DEV):
        d = lax.rem(my_i + k, N_DEV)
        for c in range(CH):
            rdma = pltpu.make_async_remote_copy(
                src_ref=x_ref.at[pl.ds(d * M, M), pl.ds(c * KC, KC)],
                dst_ref=slots_ref.at[k - 1, :, pl.ds(c * KC, KC)],
                send_sem=send_sems.at[k - 1, c],
                recv_sem=recv_sems.at[k - 1, c],
                device_id=(d,),
                device_id_type=pl.DeviceIdType.MESH,
            )
            rdma.start()
            sends.append(rdma)

    acc_ref[...] = jnp.zeros_like(acc_ref)

    def make_step(k):
        def step(c, carry):
            if k == 0:
                s = my_i
                xsrc = x_ref.at[pl.ds(my_i * M, M), pl.ds(c * KC, KC)]
            else:
                s = lax.rem(my_i - k + N_DEV, N_DEV)
                recv = pltpu.make_async_remote_copy(
                    src_ref=x_ref.at[pl.ds(s * M, M), pl.ds(c * KC, KC)],
                    dst_ref=slots_ref.at[k - 1, :, pl.ds(c * KC, KC)],
                    send_sem=send_sems.at[k - 1, c],
                    recv_sem=recv_sems.at[k - 1, c],
                    device_id=(s,),
                    device_id_type=pl.DeviceIdType.MESH,
                )
                recv.wait_recv()
                xsrc = slots_ref.at[k - 1, :, pl.ds(c * KC, KC)]
            xc = pltpu.make_async_copy(xsrc, xv_ref, dma_sems.at[0])
            xc.start()
            wc = pltpu.make_async_copy(
                w_ref.at[pl.ds(s * M + c * KC, KC), :], wv_ref,
                dma_sems.at[1])
            wc.start()
            xc.wait()
            wc.wait()
            acc_ref[...] += jnp.dot(
                xv_ref[...], wv_ref[...],
                preferred_element_type=jnp.float32)
            return carry
        return step

    for k in (0, 1, 3, 2):
        lax.fori_loop(0, CH, make_step(k), 0)

    acc_ref[...] = jnp.maximum(acc_ref[...], 0.0)
    outc = pltpu.make_async_copy(acc_ref, out_ref, out_sem)
    outc.start()
    outc.wait()

    for rdma in sends:
        rdma.wait_send()


def kernel(x, w_mat):
    k_total, m = x.shape
    n = w_mat.shape[1]

    out, _ = pl.pallas_call(
        _body,
        out_shape=[
            jax.ShapeDtypeStruct((m, n), jnp.float32),
            jax.ShapeDtypeStruct((N_DEV - 1, M, M), x.dtype),
        ],
        in_specs=[
            pl.BlockSpec(memory_space=pl.ANY),
            pl.BlockSpec(memory_space=pl.ANY),
        ],
        out_specs=[
            pl.BlockSpec(memory_space=pl.ANY),
            pl.BlockSpec(memory_space=pl.ANY),
        ],
        scratch_shapes=[
            pltpu.VMEM((M, n), jnp.float32),
            pltpu.VMEM((M, KC), x.dtype),
            pltpu.VMEM((KC, n), x.dtype),
            pltpu.SemaphoreType.DMA((N_DEV - 1, CH)),
            pltpu.SemaphoreType.DMA((N_DEV - 1, CH)),
            pltpu.SemaphoreType.DMA((2,)),
            pltpu.SemaphoreType.DMA,
        ],
        compiler_params=pltpu.CompilerParams(
            collective_id=0,
            vmem_limit_bytes=56 * 2**20,
        ),
    )(x, w_mat)
    return out
